# baseline (speedup 1.0000x reference)
import functools

import jax
import jax.numpy as jnp
from jax import lax
from jax.experimental import pallas as pl
from jax.experimental.pallas import tpu as pltpu
from jax.experimental.pallas import tpu_sc as plsc

VOCAB = 30522
MAX_POS = 512
HIDDEN = 64
B = 1024
L = 512
EPS = 1e-12

NC = 2
NS = 16
NW = NC * NS

TOKENS = B * L
CHUNK = 128
CHUNKS = TOKENS // CHUNK
UCH = 2
UNIT = UCH * CHUNK
NU_W = TOKENS // NW // UNIT
NV = HIDDEN // 16


def _ln_rows(mesh):
    @functools.partial(
        pl.kernel,
        mesh=mesh,
        compiler_params=pltpu.CompilerParams(
            use_tc_tiling_on_sc=False, needs_layout_passes=False),
        out_type=jax.ShapeDtypeStruct((B, L, HIDDEN), jnp.float32),
        scratch_types=(
            [
                pltpu.VMEM((MAX_POS, HIDDEN), jnp.float32),
                pltpu.VMEM((2, HIDDEN), jnp.float32),
                pltpu.VMEM((2, HIDDEN), jnp.float32),
                pltpu.VMEM((UCH, CHUNK), jnp.int32),
                pltpu.VMEM((UCH, CHUNK), jnp.int32),
                pltpu.VMEM((UCH, CHUNK), jnp.int32),
                pltpu.VMEM((UCH, CHUNK), jnp.int32),
                pltpu.VMEM((2 * UNIT, HIDDEN), jnp.float32),
                pltpu.VMEM((2 * UNIT, HIDDEN), jnp.float32),
                pltpu.VMEM((UNIT, 16), jnp.float32),
            ]
            + [pltpu.SemaphoreType.DMA] * (2 * UCH)
            + [pltpu.SemaphoreType.DMA] * (2 * UCH)
            + [pltpu.SemaphoreType.DMA] * 2
        ),
    )
    def k(word_hbm, ids_hbm, tt_hbm, pos_hbm, type_hbm, gb_hbm, out_hbm,
          posv, typev, gbv, idxa, idxb, tta, ttb, rowsv, outv, ttsv, *sems):
        idxv = (idxa, idxb)
        ttv = (tta, ttb)
        gsem = sems[:2 * UCH]
        ssem = sems[2 * UCH:4 * UCH]
        isem, tsem = sems[4 * UCH], sems[4 * UCH + 1]
        wid = lax.axis_index("s") * NC + lax.axis_index("c")
        cbase = wid * NU_W * UCH

        pltpu.sync_copy(pos_hbm, posv)
        pltpu.sync_copy(type_hbm, typev)
        pltpu.sync_copy(gb_hbm, gbv)

        lane = lax.iota(jnp.int32, 16)
        rots = [(lane + kk) & 15 for kk in (8, 4, 2, 1)]
        dnums = lax.GatherDimensionNumbers(
            offset_dims=(), collapsed_slice_dims=(0,), start_index_map=(0,))

        def vgather(v, ridx):
            return lax.gather(v, ridx[:, None], dnums, (1,),
                              mode=lax.GatherScatterMode.PROMISE_IN_BOUNDS)

        def allr(v):
            for ridx in rots:
                v = v + vgather(v, ridx)
            return v

        t0 = [typev[0, pl.ds(16 * h, 16)] for h in range(NV)]
        t1 = [typev[1, pl.ds(16 * h, 16)] for h in range(NV)]
        dt = [t1[h] - t0[h] for h in range(NV)]
        gvv = [gbv[0, pl.ds(16 * h, 16)] for h in range(NV)]
        bvv = [gbv[1, pl.ds(16 * h, 16)] for h in range(NV)]

        def fold_body(p, c):
            for h in range(NV):
                posv[p, pl.ds(16 * h, 16)] = (
                    posv[p, pl.ds(16 * h, 16)] + t0[h])
            return c
        lax.fori_loop(0, MAX_POS, fold_body, 0)

        def fire_gathers(par):
            for j in range(UCH):
                jj = par * UCH + j
                pltpu.make_async_copy(
                    word_hbm.at[idxv[par].at[j]],
                    rowsv.at[pl.ds(jj * CHUNK, CHUNK)], gsem[jj]).start()

        def wait_gathers(par):
            for j in range(UCH):
                jj = par * UCH + j
                pltpu.make_async_copy(
                    word_hbm.at[idxv[par].at[j]],
                    rowsv.at[pl.ds(jj * CHUNK, CHUNK)], gsem[jj]).wait()

        def drain_stores(par):
            for j in range(UCH):
                jj = par * UCH + j
                pltpu.make_async_copy(
                    outv.at[pl.ds(jj * CHUNK, CHUNK)],
                    out_hbm.at[0, pl.ds(jj * CHUNK, CHUNK)],
                    ssem[jj]).wait()

        def compute_unit(par):
            rb = par * UNIT

            def tts_body(g, c):
                ttg = ttv[par][g // (CHUNK // 16),
                               pl.ds((g % (CHUNK // 16)) * 16, 16)
                               ].astype(jnp.float32)
                for jj in range(16):
                    ttsv[g * 16 + jj, :] = vgather(
                        ttg, jnp.full((16,), jj, jnp.int32))
                return c
            lax.fori_loop(0, UNIT // 16, tts_body, 0)

            @plsc.parallel_loop(0, UNIT, 1, unroll=4)
            def _(t):
                rt = rb + t
                ttf = ttsv[t, :]
                x = []
                for h in range(NV):
                    w = rowsv[rt, pl.ds(16 * h, 16)]
                    p = posv[rt, pl.ds(16 * h, 16)]
                    x.append(w + p + ttf * dt[h])
                s = (x[0] + x[1]) + (x[2] + x[3])
                mean = allr(s) * (1.0 / HIDDEN)
                q = [xi * xi for xi in x]
                qs = (q[0] + q[1]) + (q[2] + q[3])
                var = allr(qs) * (1.0 / HIDDEN) - mean * mean + EPS
                iv = plsc.bitcast(var, jnp.int32)
                iv = 0x5F3759DF - (iv >> 1)
                y = plsc.bitcast(iv, jnp.float32)
                hvar = 0.5 * var
                y = y * (1.5 - hvar * y * y)
                y = y * (1.5 - hvar * y * y)
                ym = mean * y
                for h in range(NV):
                    outv[rt, pl.ds(16 * h, 16)] = (
                        (x[h] * y - ym) * gvv[h] + bvv[h])

        def unit_step(u, par):
            other = 1 - par

            @pl.when(u + 1 < NU_W)
            def _():
                for j in range(UCH):
                    pltpu.make_async_copy(
                        ids_hbm.at[0, pl.ds(j * CHUNK, CHUNK)],
                        idxv[other].at[j], isem).wait()
                    pltpu.make_async_copy(
                        tt_hbm.at[0, pl.ds(j * CHUNK, CHUNK)],
                        ttv[other].at[j], tsem).wait()
                fire_gathers(other)

            wait_gathers(par)

            @pl.when(u > 1)
            def _():
                drain_stores(par)

            compute_unit(par)

            @pl.when(u + 2 < NU_W)
            def _():
                un = u + 2
                bn = wid * (NU_W // 2) + (un >> 1)
                ln = (un % 2) * UNIT
                for j in range(UCH):
                    pltpu.make_async_copy(
                        ids_hbm.at[bn, pl.ds(ln + j * CHUNK, CHUNK)],
                        idxv[par].at[j], isem).start()
                    pltpu.make_async_copy(
                        tt_hbm.at[bn, pl.ds(ln + j * CHUNK, CHUNK)],
                        ttv[par].at[j], tsem).start()

            bq = wid * (NU_W // 2) + (u >> 1)
            for j in range(UCH):
                jj = par * UCH + j
                pltpu.make_async_copy(
                    outv.at[pl.ds(jj * CHUNK, CHUNK)],
                    out_hbm.at[bq, pl.ds(jj * CHUNK, CHUNK)],
                    ssem[jj]).start()

        b0 = wid * (NU_W // 2)
        for j in range(UCH):
            pltpu.sync_copy(ids_hbm.at[b0, pl.ds(j * CHUNK, CHUNK)],
                            idxv[0].at[j])
            pltpu.sync_copy(tt_hbm.at[b0, pl.ds(j * CHUNK, CHUNK)],
                            ttv[0].at[j])
        fire_gathers(0)
        for j in range(UCH):
            pltpu.make_async_copy(
                ids_hbm.at[b0, pl.ds(UNIT + j * CHUNK, CHUNK)],
                idxv[1].at[j], isem).start()
            pltpu.make_async_copy(
                tt_hbm.at[b0, pl.ds(UNIT + j * CHUNK, CHUNK)],
                ttv[1].at[j], tsem).start()

        def pair_body(i, c):
            unit_step(i * 2, 0)
            unit_step(i * 2 + 1, 1)
            return c

        lax.fori_loop(0, NU_W // 2, pair_body, 0)

        drain_stores(0)
        drain_stores(1)

    return k


def kernel(input_ids, token_type_ids, word_emb, pos_emb, type_emb, gamma,
           beta):
    ids = input_ids.astype(jnp.int32)
    tt = token_type_ids.astype(jnp.int32)
    gb = jnp.stack([gamma, beta]).astype(jnp.float32)
    mesh = plsc.VectorSubcoreMesh(core_axis_name="c", subcore_axis_name="s")
    return _ln_rows(mesh)(word_emb, ids, tt, pos_emb, type_emb, gb)


# speedup vs baseline: 1.0014x; 1.0014x over previous
import functools

import jax
import jax.numpy as jnp
from jax import lax
from jax.experimental import pallas as pl
from jax.experimental.pallas import tpu as pltpu
from jax.experimental.pallas import tpu_sc as plsc

VOCAB = 30522
MAX_POS = 512
HIDDEN = 64
B = 1024
L = 512
EPS = 1e-12

NC = 2
NS = 16
NW = NC * NS

TOKENS = B * L
CHUNK = 128
CHUNKS = TOKENS // CHUNK
UCH = 2
UNIT = UCH * CHUNK
NU_W = TOKENS // NW // UNIT
NV = HIDDEN // 16


def _ln_rows(mesh):
    @functools.partial(
        pl.kernel,
        mesh=mesh,
        compiler_params=pltpu.CompilerParams(
            use_tc_tiling_on_sc=False, needs_layout_passes=False),
        out_type=jax.ShapeDtypeStruct((TOKENS // 2, 2 * HIDDEN), jnp.float32),
        scratch_types=(
            [
                pltpu.VMEM((MAX_POS, HIDDEN), jnp.float32),
                pltpu.VMEM((2, HIDDEN), jnp.float32),
                pltpu.VMEM((2, HIDDEN), jnp.float32),
                pltpu.VMEM((UCH, CHUNK), jnp.int32),
                pltpu.VMEM((UCH, CHUNK), jnp.int32),
                pltpu.VMEM((UCH, CHUNK), jnp.int32),
                pltpu.VMEM((UCH, CHUNK), jnp.int32),
                pltpu.VMEM((2 * UNIT, HIDDEN), jnp.float32),
                pltpu.VMEM((UNIT, 2 * HIDDEN), jnp.float32),
                pltpu.VMEM((UNIT, 16), jnp.float32),
            ]
            + [pltpu.SemaphoreType.DMA] * (2 * UCH)
            + [pltpu.SemaphoreType.DMA] * (2 * UCH)
            + [pltpu.SemaphoreType.DMA] * 2
        ),
    )
    def k(word_hbm, ids_hbm, tt_hbm, pos_hbm, type_hbm, gb_hbm, out_hbm,
          posv, typev, gbv, idxa, idxb, tta, ttb, rowsv, outv, ttsv, *sems):
        idxv = (idxa, idxb)
        ttv = (tta, ttb)
        gsem = sems[:2 * UCH]
        ssem = sems[2 * UCH:4 * UCH]
        isem, tsem = sems[4 * UCH], sems[4 * UCH + 1]
        wid = lax.axis_index("s") * NC + lax.axis_index("c")
        cbase = wid * NU_W * UCH

        pltpu.sync_copy(pos_hbm, posv)
        pltpu.sync_copy(type_hbm, typev)
        pltpu.sync_copy(gb_hbm, gbv)

        lane = lax.iota(jnp.int32, 16)
        rots = [(lane + kk) & 15 for kk in (8, 4, 2, 1)]
        dnums = lax.GatherDimensionNumbers(
            offset_dims=(), collapsed_slice_dims=(0,), start_index_map=(0,))

        def vgather(v, ridx):
            return lax.gather(v, ridx[:, None], dnums, (1,),
                              mode=lax.GatherScatterMode.PROMISE_IN_BOUNDS)

        def allr(v):
            for ridx in rots:
                v = v + vgather(v, ridx)
            return v

        t0 = [typev[0, pl.ds(16 * h, 16)] for h in range(NV)]
        t1 = [typev[1, pl.ds(16 * h, 16)] for h in range(NV)]
        dt = [t1[h] - t0[h] for h in range(NV)]
        gvv = [gbv[0, pl.ds(16 * h, 16)] for h in range(NV)]
        bvv = [gbv[1, pl.ds(16 * h, 16)] for h in range(NV)]

        def fold_body(p, c):
            for h in range(NV):
                posv[p, pl.ds(16 * h, 16)] = (
                    posv[p, pl.ds(16 * h, 16)] + t0[h])
            return c
        lax.fori_loop(0, MAX_POS, fold_body, 0)

        def fire_gathers(par):
            for j in range(UCH):
                jj = par * UCH + j
                pltpu.make_async_copy(
                    word_hbm.at[idxv[par].at[j]],
                    rowsv.at[pl.ds(jj * CHUNK, CHUNK)], gsem[jj]).start()

        def wait_gathers(par):
            for j in range(UCH):
                jj = par * UCH + j
                pltpu.make_async_copy(
                    word_hbm.at[idxv[par].at[j]],
                    rowsv.at[pl.ds(jj * CHUNK, CHUNK)], gsem[jj]).wait()

        def drain_stores(par):
            for j in range(UCH):
                jj = par * UCH + j
                pltpu.make_async_copy(
                    outv.at[pl.ds(jj * (CHUNK // 2), CHUNK // 2)],
                    out_hbm.at[pl.ds(jj * (CHUNK // 2), CHUNK // 2)],
                    ssem[jj]).wait()

        def compute_unit(par):
            rb = par * UNIT

            def tts_body(g, c):
                ttg = ttv[par][g // (CHUNK // 16),
                               pl.ds((g % (CHUNK // 16)) * 16, 16)
                               ].astype(jnp.float32)
                for jj in range(16):
                    ttsv[g * 16 + jj, :] = vgather(
                        ttg, jnp.full((16,), jj, jnp.int32))
                return c
            lax.fori_loop(0, UNIT // 16, tts_body, 0)

            @plsc.parallel_loop(0, UNIT, 1, unroll=4)
            def _(t):
                rt = rb + t
                ttf = ttsv[t, :]
                x = []
                for h in range(NV):
                    w = rowsv[rt, pl.ds(16 * h, 16)]
                    p = posv[rt, pl.ds(16 * h, 16)]
                    x.append(w + p + ttf * dt[h])
                s = (x[0] + x[1]) + (x[2] + x[3])
                mean = allr(s) * (1.0 / HIDDEN)
                q = [xi * xi for xi in x]
                qs = (q[0] + q[1]) + (q[2] + q[3])
                var = allr(qs) * (1.0 / HIDDEN) - mean * mean + EPS
                iv = plsc.bitcast(var, jnp.int32)
                iv = 0x5F3759DF - (iv >> 1)
                y = plsc.bitcast(iv, jnp.float32)
                hvar = 0.5 * var
                y = y * (1.5 - hvar * y * y)
                y = y * (1.5 - hvar * y * y)
                ym = mean * y
                ro = rt >> 1
                co = (rt & 1) * HIDDEN
                for h in range(NV):
                    outv[ro, pl.ds(co + 16 * h, 16)] = (
                        (x[h] * y - ym) * gvv[h] + bvv[h])

        def unit_step(u, par):
            other = 1 - par

            @pl.when(u + 1 < NU_W)
            def _():
                for j in range(UCH):
                    pltpu.make_async_copy(
                        ids_hbm.at[0, pl.ds(j * CHUNK, CHUNK)],
                        idxv[other].at[j], isem).wait()
                    pltpu.make_async_copy(
                        tt_hbm.at[0, pl.ds(j * CHUNK, CHUNK)],
                        ttv[other].at[j], tsem).wait()
                fire_gathers(other)

            wait_gathers(par)

            @pl.when(u > 1)
            def _():
                drain_stores(par)

            compute_unit(par)

            @pl.when(u + 2 < NU_W)
            def _():
                un = u + 2
                bn = wid * (NU_W // 2) + (un >> 1)
                ln = (un % 2) * UNIT
                for j in range(UCH):
                    pltpu.make_async_copy(
                        ids_hbm.at[bn, pl.ds(ln + j * CHUNK, CHUNK)],
                        idxv[par].at[j], isem).start()
                    pltpu.make_async_copy(
                        tt_hbm.at[bn, pl.ds(ln + j * CHUNK, CHUNK)],
                        ttv[par].at[j], tsem).start()

            t0g = (wid * NU_W + u) * UNIT
            for j in range(UCH):
                jj = par * UCH + j
                pltpu.make_async_copy(
                    outv.at[pl.ds(jj * (CHUNK // 2), CHUNK // 2)],
                    out_hbm.at[pl.ds((t0g + j * CHUNK) // 2, CHUNK // 2)],
                    ssem[jj]).start()

        b0 = wid * (NU_W // 2)
        for j in range(UCH):
            pltpu.sync_copy(ids_hbm.at[b0, pl.ds(j * CHUNK, CHUNK)],
                            idxv[0].at[j])
            pltpu.sync_copy(tt_hbm.at[b0, pl.ds(j * CHUNK, CHUNK)],
                            ttv[0].at[j])
        fire_gathers(0)
        for j in range(UCH):
            pltpu.make_async_copy(
                ids_hbm.at[b0, pl.ds(UNIT + j * CHUNK, CHUNK)],
                idxv[1].at[j], isem).start()
            pltpu.make_async_copy(
                tt_hbm.at[b0, pl.ds(UNIT + j * CHUNK, CHUNK)],
                ttv[1].at[j], tsem).start()

        def pair_body(i, c):
            unit_step(i * 2, 0)
            unit_step(i * 2 + 1, 1)
            return c

        lax.fori_loop(0, NU_W // 2, pair_body, 0)

        drain_stores(0)
        drain_stores(1)

    return k


def kernel(input_ids, token_type_ids, word_emb, pos_emb, type_emb, gamma,
           beta):
    ids = input_ids.astype(jnp.int32)
    tt = token_type_ids.astype(jnp.int32)
    gb = jnp.stack([gamma, beta]).astype(jnp.float32)
    mesh = plsc.VectorSubcoreMesh(core_axis_name="c", subcore_axis_name="s")
    out = _ln_rows(mesh)(word_emb, ids, tt, pos_emb, type_emb, gb)
    return out.reshape(B, L, HIDDEN)


# speedup vs baseline: 1.0668x; 1.0653x over previous
import functools

import jax
import jax.numpy as jnp
from jax import lax
from jax.experimental import pallas as pl
from jax.experimental.pallas import tpu as pltpu
from jax.experimental.pallas import tpu_sc as plsc

VOCAB = 30522
MAX_POS = 512
HIDDEN = 64
B = 1024
L = 512
EPS = 1e-12

NC = 2
NS = 16
NW = NC * NS

TOKENS = B * L
CHUNK = 128
CHUNKS = TOKENS // CHUNK
UCH = 2
UNIT = UCH * CHUNK
NU_W = TOKENS // NW // UNIT
NV = HIDDEN // 16


def _ln_rows(mesh):
    @functools.partial(
        pl.kernel,
        mesh=mesh,
        compiler_params=pltpu.CompilerParams(
            use_tc_tiling_on_sc=False, needs_layout_passes=False),
        out_type=jax.ShapeDtypeStruct((B, HIDDEN, L), jnp.float32),
        scratch_types=(
            [
                pltpu.VMEM((MAX_POS, HIDDEN), jnp.float32),
                pltpu.VMEM((2, HIDDEN), jnp.float32),
                pltpu.VMEM((2, HIDDEN), jnp.float32),
                pltpu.VMEM((UCH, CHUNK), jnp.int32),
                pltpu.VMEM((UCH, CHUNK), jnp.int32),
                pltpu.VMEM((UCH, CHUNK), jnp.int32),
                pltpu.VMEM((UCH, CHUNK), jnp.int32),
                pltpu.VMEM((2 * UNIT, HIDDEN), jnp.float32),
                pltpu.VMEM((2 * UCH * HIDDEN, CHUNK + 1), jnp.float32),
                pltpu.VMEM((UNIT, 16), jnp.float32),
            ]
            + [pltpu.SemaphoreType.DMA] * (2 * UCH)
            + [pltpu.SemaphoreType.DMA] * (2 * UCH)
            + [pltpu.SemaphoreType.DMA] * 2
        ),
    )
    def k(word_hbm, ids_hbm, tt_hbm, pos_hbm, type_hbm, gb_hbm, out_hbm,
          posv, typev, gbv, idxa, idxb, tta, ttb, rowsv, outv, ttsv, *sems):
        idxv = (idxa, idxb)
        ttv = (tta, ttb)
        gsem = sems[:2 * UCH]
        ssem = sems[2 * UCH:4 * UCH]
        isem, tsem = sems[4 * UCH], sems[4 * UCH + 1]
        wid = lax.axis_index("s") * NC + lax.axis_index("c")
        cbase = wid * NU_W * UCH

        pltpu.sync_copy(pos_hbm, posv)
        pltpu.sync_copy(type_hbm, typev)
        pltpu.sync_copy(gb_hbm, gbv)

        lane = lax.iota(jnp.int32, 16)
        rots = [(lane + kk) & 15 for kk in (8, 4, 2, 1)]
        hrow = [lane + 16 * h for h in range(NV)]
        dnums = lax.GatherDimensionNumbers(
            offset_dims=(), collapsed_slice_dims=(0,), start_index_map=(0,))

        def vgather(v, ridx):
            return lax.gather(v, ridx[:, None], dnums, (1,),
                              mode=lax.GatherScatterMode.PROMISE_IN_BOUNDS)

        def allr(v):
            for ridx in rots:
                v = v + vgather(v, ridx)
            return v

        t0 = [typev[0, pl.ds(16 * h, 16)] for h in range(NV)]
        t1 = [typev[1, pl.ds(16 * h, 16)] for h in range(NV)]
        dt = [t1[h] - t0[h] for h in range(NV)]
        gvv = [gbv[0, pl.ds(16 * h, 16)] for h in range(NV)]
        bvv = [gbv[1, pl.ds(16 * h, 16)] for h in range(NV)]

        def fold_body(p, c):
            for h in range(NV):
                posv[p, pl.ds(16 * h, 16)] = (
                    posv[p, pl.ds(16 * h, 16)] + t0[h])
            return c
        lax.fori_loop(0, MAX_POS, fold_body, 0)

        def fire_gathers(par):
            for j in range(UCH):
                jj = par * UCH + j
                pltpu.make_async_copy(
                    word_hbm.at[idxv[par].at[j]],
                    rowsv.at[pl.ds(jj * CHUNK, CHUNK)], gsem[jj]).start()

        def wait_gathers(par):
            for j in range(UCH):
                jj = par * UCH + j
                pltpu.make_async_copy(
                    word_hbm.at[idxv[par].at[j]],
                    rowsv.at[pl.ds(jj * CHUNK, CHUNK)], gsem[jj]).wait()

        def drain_stores(par):
            for j in range(UCH):
                jj = par * UCH + j
                pltpu.make_async_copy(
                    outv.at[pl.ds(jj * HIDDEN, HIDDEN), pl.ds(0, CHUNK)],
                    out_hbm.at[0, :, pl.ds(jj * CHUNK, CHUNK)],
                    ssem[jj]).wait()

        def compute_unit(par):
            rb = par * UNIT

            def tts_body(g, c):
                ttg = ttv[par][g // (CHUNK // 16),
                               pl.ds((g % (CHUNK // 16)) * 16, 16)
                               ].astype(jnp.float32)
                for jj in range(16):
                    ttsv[g * 16 + jj, :] = vgather(
                        ttg, jnp.full((16,), jj, jnp.int32))
                return c
            lax.fori_loop(0, UNIT // 16, tts_body, 0)

            @plsc.parallel_loop(0, UNIT, 1, unroll=4)
            def _(t):
                rt = rb + t
                ttf = ttsv[t, :]
                x = []
                for h in range(NV):
                    w = rowsv[rt, pl.ds(16 * h, 16)]
                    p = posv[rt, pl.ds(16 * h, 16)]
                    x.append(w + p + ttf * dt[h])
                s = (x[0] + x[1]) + (x[2] + x[3])
                mean = allr(s) * (1.0 / HIDDEN)
                q = [xi * xi for xi in x]
                qs = (q[0] + q[1]) + (q[2] + q[3])
                var = allr(qs) * (1.0 / HIDDEN) - mean * mean + EPS
                iv = plsc.bitcast(var, jnp.int32)
                iv = 0x5F3759DF - (iv >> 1)
                y = plsc.bitcast(iv, jnp.float32)
                hvar = 0.5 * var
                y = y * (1.5 - hvar * y * y)
                y = y * (1.5 - hvar * y * y)
                ym = mean * y
                rbase = (par * UCH + (t >> 7)) * HIDDEN
                lcol = lane * 0 + (t & (CHUNK - 1))
                for h in range(NV):
                    o = (x[h] * y - ym) * gvv[h] + bvv[h]
                    plsc.store_scatter(
                        outv, [rbase + hrow[h], lcol], o)

        def unit_step(u, par):
            other = 1 - par

            @pl.when(u + 1 < NU_W)
            def _():
                for j in range(UCH):
                    pltpu.make_async_copy(
                        ids_hbm.at[0, pl.ds(j * CHUNK, CHUNK)],
                        idxv[other].at[j], isem).wait()
                    pltpu.make_async_copy(
                        tt_hbm.at[0, pl.ds(j * CHUNK, CHUNK)],
                        ttv[other].at[j], tsem).wait()
                fire_gathers(other)

            wait_gathers(par)

            @pl.when(u > 1)
            def _():
                drain_stores(par)

            compute_unit(par)

            @pl.when(u + 2 < NU_W)
            def _():
                un = u + 2
                bn = wid * (NU_W // 2) + (un >> 1)
                ln = (un % 2) * UNIT
                for j in range(UCH):
                    pltpu.make_async_copy(
                        ids_hbm.at[bn, pl.ds(ln + j * CHUNK, CHUNK)],
                        idxv[par].at[j], isem).start()
                    pltpu.make_async_copy(
                        tt_hbm.at[bn, pl.ds(ln + j * CHUNK, CHUNK)],
                        ttv[par].at[j], tsem).start()

            bq = wid * (NU_W // 2) + (u >> 1)
            for j in range(UCH):
                jj = par * UCH + j
                pltpu.make_async_copy(
                    outv.at[pl.ds(jj * HIDDEN, HIDDEN), pl.ds(0, CHUNK)],
                    out_hbm.at[bq, :, pl.ds(jj * CHUNK, CHUNK)],
                    ssem[jj]).start()

        b0 = wid * (NU_W // 2)
        for j in range(UCH):
            pltpu.sync_copy(ids_hbm.at[b0, pl.ds(j * CHUNK, CHUNK)],
                            idxv[0].at[j])
            pltpu.sync_copy(tt_hbm.at[b0, pl.ds(j * CHUNK, CHUNK)],
                            ttv[0].at[j])
        fire_gathers(0)
        for j in range(UCH):
            pltpu.make_async_copy(
                ids_hbm.at[b0, pl.ds(UNIT + j * CHUNK, CHUNK)],
                idxv[1].at[j], isem).start()
            pltpu.make_async_copy(
                tt_hbm.at[b0, pl.ds(UNIT + j * CHUNK, CHUNK)],
                ttv[1].at[j], tsem).start()

        def pair_body(i, c):
            unit_step(i * 2, 0)
            unit_step(i * 2 + 1, 1)
            return c

        lax.fori_loop(0, NU_W // 2, pair_body, 0)

        drain_stores(0)
        drain_stores(1)

    return k


def kernel(input_ids, token_type_ids, word_emb, pos_emb, type_emb, gamma,
           beta):
    ids = input_ids.astype(jnp.int32)
    tt = token_type_ids.astype(jnp.int32)
    gb = jnp.stack([gamma, beta]).astype(jnp.float32)
    mesh = plsc.VectorSubcoreMesh(core_axis_name="c", subcore_axis_name="s")
    out = _ln_rows(mesh)(word_emb, ids, tt, pos_emb, type_emb, gb)
    return jnp.swapaxes(out, 1, 2)


# speedup vs baseline: 1.3604x; 1.2753x over previous
import functools

import jax
import jax.numpy as jnp
from jax import lax
from jax.experimental import pallas as pl
from jax.experimental.pallas import tpu as pltpu
from jax.experimental.pallas import tpu_sc as plsc

VOCAB = 30522
MAX_POS = 512
HIDDEN = 64
B = 1024
L = 512
EPS = 1e-12

NC = 2
NS = 16
NW = NC * NS

TOKENS = B * L
CHUNK = 128
CHUNKS = TOKENS // CHUNK
UCH = 2
UNIT = UCH * CHUNK
NU_W = TOKENS // NW // UNIT
NV = HIDDEN // 16


def _ln_rows(mesh):
    @functools.partial(
        pl.kernel,
        mesh=mesh,
        compiler_params=pltpu.CompilerParams(
            use_tc_tiling_on_sc=False, needs_layout_passes=False),
        out_type=jax.ShapeDtypeStruct(
            (B, HIDDEN // 8, L // CHUNK, 8, CHUNK), jnp.float32),
        scratch_types=(
            [
                pltpu.VMEM((MAX_POS, HIDDEN), jnp.float32),
                pltpu.VMEM((2, HIDDEN), jnp.float32),
                pltpu.VMEM((2, HIDDEN), jnp.float32),
                pltpu.VMEM((UCH, CHUNK), jnp.int32),
                pltpu.VMEM((UCH, CHUNK), jnp.int32),
                pltpu.VMEM((UCH, CHUNK), jnp.int32),
                pltpu.VMEM((UCH, CHUNK), jnp.int32),
                pltpu.VMEM((2 * UNIT, HIDDEN), jnp.float32),
                pltpu.VMEM((2 * UCH * HIDDEN, CHUNK + 1), jnp.float32),
                pltpu.VMEM((UNIT, 16), jnp.float32),
            ]
            + [pltpu.SemaphoreType.DMA] * (2 * UCH)
            + [pltpu.SemaphoreType.DMA] * (2 * UCH)
            + [pltpu.SemaphoreType.DMA] * 2
        ),
    )
    def k(word_hbm, ids_hbm, tt_hbm, pos_hbm, type_hbm, gb_hbm, out_hbm,
          posv, typev, gbv, idxa, idxb, tta, ttb, rowsv, outv, ttsv, *sems):
        idxv = (idxa, idxb)
        ttv = (tta, ttb)
        gsem = sems[:2 * UCH]
        ssem = sems[2 * UCH:4 * UCH]
        isem, tsem = sems[4 * UCH], sems[4 * UCH + 1]
        wid = lax.axis_index("s") * NC + lax.axis_index("c")
        cbase = wid * NU_W * UCH

        pltpu.sync_copy(pos_hbm, posv)
        pltpu.sync_copy(type_hbm, typev)
        pltpu.sync_copy(gb_hbm, gbv)

        lane = lax.iota(jnp.int32, 16)
        rots = [(lane + kk) & 15 for kk in (8, 4, 2, 1)]
        hrow = [lane + 16 * h for h in range(NV)]
        dnums = lax.GatherDimensionNumbers(
            offset_dims=(), collapsed_slice_dims=(0,), start_index_map=(0,))

        def vgather(v, ridx):
            return lax.gather(v, ridx[:, None], dnums, (1,),
                              mode=lax.GatherScatterMode.PROMISE_IN_BOUNDS)

        def allr(v):
            for ridx in rots:
                v = v + vgather(v, ridx)
            return v

        t0 = [typev[0, pl.ds(16 * h, 16)] for h in range(NV)]
        t1 = [typev[1, pl.ds(16 * h, 16)] for h in range(NV)]
        dt = [t1[h] - t0[h] for h in range(NV)]
        gvv = [gbv[0, pl.ds(16 * h, 16)] for h in range(NV)]
        bvv = [gbv[1, pl.ds(16 * h, 16)] for h in range(NV)]

        def fold_body(p, c):
            for h in range(NV):
                posv[p, pl.ds(16 * h, 16)] = (
                    posv[p, pl.ds(16 * h, 16)] + t0[h])
            return c
        lax.fori_loop(0, MAX_POS, fold_body, 0)

        def fire_gathers(par):
            for j in range(UCH):
                jj = par * UCH + j
                pltpu.make_async_copy(
                    word_hbm.at[idxv[par].at[j]],
                    rowsv.at[pl.ds(jj * CHUNK, CHUNK)], gsem[jj]).start()

        def wait_gathers(par):
            for j in range(UCH):
                jj = par * UCH + j
                pltpu.make_async_copy(
                    word_hbm.at[idxv[par].at[j]],
                    rowsv.at[pl.ds(jj * CHUNK, CHUNK)], gsem[jj]).wait()

        def drain_stores(par):
            for j in range(UCH):
                jj = par * UCH + j
                for hr in range(HIDDEN // 8):
                    pltpu.make_async_copy(
                        outv.at[pl.ds(jj * HIDDEN + 8 * hr, 8),
                                pl.ds(0, CHUNK)],
                        out_hbm.at[0, hr, jj], ssem[jj]).wait()

        def compute_unit(par):
            rb = par * UNIT

            def tts_body(g, c):
                ttg = ttv[par][g // (CHUNK // 16),
                               pl.ds((g % (CHUNK // 16)) * 16, 16)
                               ].astype(jnp.float32)
                for jj in range(16):
                    ttsv[g * 16 + jj, :] = vgather(
                        ttg, jnp.full((16,), jj, jnp.int32))
                return c
            lax.fori_loop(0, UNIT // 16, tts_body, 0)

            @plsc.parallel_loop(0, UNIT, 1, unroll=4)
            def _(t):
                rt = rb + t
                ttf = ttsv[t, :]
                x = []
                for h in range(NV):
                    w = rowsv[rt, pl.ds(16 * h, 16)]
                    p = posv[rt, pl.ds(16 * h, 16)]
                    x.append(w + p + ttf * dt[h])
                s = (x[0] + x[1]) + (x[2] + x[3])
                mean = allr(s) * (1.0 / HIDDEN)
                q = [xi * xi for xi in x]
                qs = (q[0] + q[1]) + (q[2] + q[3])
                var = allr(qs) * (1.0 / HIDDEN) - mean * mean + EPS
                iv = plsc.bitcast(var, jnp.int32)
                iv = 0x5F3759DF - (iv >> 1)
                y = plsc.bitcast(iv, jnp.float32)
                hvar = 0.5 * var
                y = y * (1.5 - hvar * y * y)
                y = y * (1.5 - hvar * y * y)
                ym = mean * y
                rbase = (par * UCH + (t >> 7)) * HIDDEN
                lcol = lane * 0 + (t & (CHUNK - 1))
                for h in range(NV):
                    o = (x[h] * y - ym) * gvv[h] + bvv[h]
                    plsc.store_scatter(
                        outv, [rbase + hrow[h], lcol], o)

        def unit_step(u, par):
            other = 1 - par

            @pl.when(u + 1 < NU_W)
            def _():
                for j in range(UCH):
                    pltpu.make_async_copy(
                        ids_hbm.at[0, pl.ds(j * CHUNK, CHUNK)],
                        idxv[other].at[j], isem).wait()
                    pltpu.make_async_copy(
                        tt_hbm.at[0, pl.ds(j * CHUNK, CHUNK)],
                        ttv[other].at[j], tsem).wait()
                fire_gathers(other)

            wait_gathers(par)

            @pl.when(u > 1)
            def _():
                drain_stores(par)

            compute_unit(par)

            @pl.when(u + 2 < NU_W)
            def _():
                un = u + 2
                bn = wid * (NU_W // 2) + (un >> 1)
                ln = (un % 2) * UNIT
                for j in range(UCH):
                    pltpu.make_async_copy(
                        ids_hbm.at[bn, pl.ds(ln + j * CHUNK, CHUNK)],
                        idxv[par].at[j], isem).start()
                    pltpu.make_async_copy(
                        tt_hbm.at[bn, pl.ds(ln + j * CHUNK, CHUNK)],
                        ttv[par].at[j], tsem).start()

            bq = wid * (NU_W // 2) + (u >> 1)
            for j in range(UCH):
                jj = par * UCH + j
                for hr in range(HIDDEN // 8):
                    pltpu.make_async_copy(
                        outv.at[pl.ds(jj * HIDDEN + 8 * hr, 8),
                                pl.ds(0, CHUNK)],
                        out_hbm.at[bq, hr, jj], ssem[jj]).start()

        b0 = wid * (NU_W // 2)
        for j in range(UCH):
            pltpu.sync_copy(ids_hbm.at[b0, pl.ds(j * CHUNK, CHUNK)],
                            idxv[0].at[j])
            pltpu.sync_copy(tt_hbm.at[b0, pl.ds(j * CHUNK, CHUNK)],
                            ttv[0].at[j])
        fire_gathers(0)
        for j in range(UCH):
            pltpu.make_async_copy(
                ids_hbm.at[b0, pl.ds(UNIT + j * CHUNK, CHUNK)],
                idxv[1].at[j], isem).start()
            pltpu.make_async_copy(
                tt_hbm.at[b0, pl.ds(UNIT + j * CHUNK, CHUNK)],
                ttv[1].at[j], tsem).start()

        def pair_body(i, c):
            unit_step(i * 2, 0)
            unit_step(i * 2 + 1, 1)
            return c

        lax.fori_loop(0, NU_W // 2, pair_body, 0)

        drain_stores(0)
        drain_stores(1)

    return k


def kernel(input_ids, token_type_ids, word_emb, pos_emb, type_emb, gamma,
           beta):
    ids = input_ids.astype(jnp.int32)
    tt = token_type_ids.astype(jnp.int32)
    gb = jnp.stack([gamma, beta]).astype(jnp.float32)
    mesh = plsc.VectorSubcoreMesh(core_axis_name="c", subcore_axis_name="s")
    out = _ln_rows(mesh)(word_emb, ids, tt, pos_emb, type_emb, gb)
    return out.transpose(0, 2, 4, 1, 3).reshape(B, L, HIDDEN)


# speedup vs baseline: 1.3799x; 1.0143x over previous
import functools

import jax
import jax.numpy as jnp
from jax import lax
from jax.experimental import pallas as pl
from jax.experimental.pallas import tpu as pltpu
from jax.experimental.pallas import tpu_sc as plsc

VOCAB = 30522
MAX_POS = 512
HIDDEN = 64
B = 1024
L = 512
EPS = 1e-12

NC = 2
NS = 16
NW = NC * NS

TOKENS = B * L
CHUNK = 128
CHUNKS = TOKENS // CHUNK
UCH = 2
UNIT = UCH * CHUNK
NU_W = TOKENS // NW // UNIT
NV = HIDDEN // 16


def _ln_rows(mesh):
    @functools.partial(
        pl.kernel,
        mesh=mesh,
        compiler_params=pltpu.CompilerParams(
            use_tc_tiling_on_sc=False, needs_layout_passes=False),
        out_type=jax.ShapeDtypeStruct(
            (B, HIDDEN // 8, L // CHUNK, 8, CHUNK), jnp.float32),
        scratch_types=(
            [
                pltpu.VMEM((MAX_POS, HIDDEN), jnp.float32),
                pltpu.VMEM((2, HIDDEN), jnp.float32),
                pltpu.VMEM((2, HIDDEN), jnp.float32),
                pltpu.VMEM((UCH, CHUNK), jnp.int32),
                pltpu.VMEM((UCH, CHUNK), jnp.int32),
                pltpu.VMEM((UCH, CHUNK), jnp.int32),
                pltpu.VMEM((UCH, CHUNK), jnp.int32),
                pltpu.VMEM((2 * UNIT, HIDDEN), jnp.float32),
                pltpu.VMEM((2 * UCH * HIDDEN, CHUNK + 1), jnp.float32),
                pltpu.VMEM((UNIT, 16), jnp.float32),
            ]
            + [pltpu.SemaphoreType.DMA] * (2 * UCH)
            + [pltpu.SemaphoreType.DMA] * (2 * UCH)
            + [pltpu.SemaphoreType.DMA] * 2
        ),
    )
    def k(word_hbm, ids_hbm, tt_hbm, pos_hbm, type_hbm, gb_hbm, out_hbm,
          posv, typev, gbv, idxa, idxb, tta, ttb, rowsv, outv, ttsv, *sems):
        idxv = (idxa, idxb)
        ttv = (tta, ttb)
        gsem = sems[:2 * UCH]
        ssem = sems[2 * UCH:4 * UCH]
        isem, tsem = sems[4 * UCH], sems[4 * UCH + 1]
        wid = lax.axis_index("s") * NC + lax.axis_index("c")
        cbase = wid * NU_W * UCH

        pltpu.sync_copy(pos_hbm, posv)
        pltpu.sync_copy(type_hbm, typev)
        pltpu.sync_copy(gb_hbm, gbv)

        lane = lax.iota(jnp.int32, 16)
        rots = [(lane + kk) & 15 for kk in (8, 4, 2, 1)]
        hrow = [lane + 16 * h for h in range(NV)]
        dnums = lax.GatherDimensionNumbers(
            offset_dims=(), collapsed_slice_dims=(0,), start_index_map=(0,))

        def vgather(v, ridx):
            return lax.gather(v, ridx[:, None], dnums, (1,),
                              mode=lax.GatherScatterMode.PROMISE_IN_BOUNDS)

        def allr(v):
            for ridx in rots:
                v = v + vgather(v, ridx)
            return v

        t0 = [typev[0, pl.ds(16 * h, 16)] for h in range(NV)]
        t1 = [typev[1, pl.ds(16 * h, 16)] for h in range(NV)]
        dt = [t1[h] - t0[h] for h in range(NV)]
        gvv = [gbv[0, pl.ds(16 * h, 16)] for h in range(NV)]
        bvv = [gbv[1, pl.ds(16 * h, 16)] for h in range(NV)]

        def fold_body(p, c):
            for h in range(NV):
                posv[p, pl.ds(16 * h, 16)] = (
                    posv[p, pl.ds(16 * h, 16)] + t0[h])
            return c
        lax.fori_loop(0, MAX_POS, fold_body, 0)

        def fire_gathers(par):
            for j in range(UCH):
                jj = par * UCH + j
                pltpu.make_async_copy(
                    word_hbm.at[idxv[par].at[j]],
                    rowsv.at[pl.ds(jj * CHUNK, CHUNK)], gsem[jj]).start()

        def wait_gathers(par):
            for j in range(UCH):
                jj = par * UCH + j
                pltpu.make_async_copy(
                    word_hbm.at[idxv[par].at[j]],
                    rowsv.at[pl.ds(jj * CHUNK, CHUNK)], gsem[jj]).wait()

        def drain_stores(par):
            for j in range(UCH):
                jj = par * UCH + j
                pltpu.make_async_copy(
                    outv.at[pl.ds(jj * HIDDEN, HIDDEN), pl.ds(0, CHUNK)],
                    out_hbm.at[0, :, jj], ssem[jj]).wait()

        def compute_unit(par):
            rb = par * UNIT

            def tts_body(g, c):
                ttg = ttv[par][g // (CHUNK // 16),
                               pl.ds((g % (CHUNK // 16)) * 16, 16)
                               ].astype(jnp.float32)
                for jj in range(16):
                    ttsv[g * 16 + jj, :] = vgather(
                        ttg, jnp.full((16,), jj, jnp.int32))
                return c
            lax.fori_loop(0, UNIT // 16, tts_body, 0)

            @plsc.parallel_loop(0, UNIT, 1, unroll=4)
            def _(t):
                rt = rb + t
                ttf = ttsv[t, :]
                x = []
                for h in range(NV):
                    w = rowsv[rt, pl.ds(16 * h, 16)]
                    p = posv[rt, pl.ds(16 * h, 16)]
                    x.append(w + p + ttf * dt[h])
                s = (x[0] + x[1]) + (x[2] + x[3])
                mean = allr(s) * (1.0 / HIDDEN)
                q = [xi * xi for xi in x]
                qs = (q[0] + q[1]) + (q[2] + q[3])
                var = allr(qs) * (1.0 / HIDDEN) - mean * mean + EPS
                iv = plsc.bitcast(var, jnp.int32)
                iv = 0x5F3759DF - (iv >> 1)
                y = plsc.bitcast(iv, jnp.float32)
                hvar = 0.5 * var
                y = y * (1.5 - hvar * y * y)
                y = y * (1.5 - hvar * y * y)
                ym = mean * y
                rbase = (par * UCH + (t >> 7)) * HIDDEN
                lcol = lane * 0 + (t & (CHUNK - 1))
                for h in range(NV):
                    o = (x[h] * y - ym) * gvv[h] + bvv[h]
                    plsc.store_scatter(
                        outv, [rbase + hrow[h], lcol], o)

        def unit_step(u, par):
            other = 1 - par

            @pl.when(u + 1 < NU_W)
            def _():
                for j in range(UCH):
                    pltpu.make_async_copy(
                        ids_hbm.at[0, pl.ds(j * CHUNK, CHUNK)],
                        idxv[other].at[j], isem).wait()
                    pltpu.make_async_copy(
                        tt_hbm.at[0, pl.ds(j * CHUNK, CHUNK)],
                        ttv[other].at[j], tsem).wait()
                fire_gathers(other)

            wait_gathers(par)

            @pl.when(u > 1)
            def _():
                drain_stores(par)

            compute_unit(par)

            @pl.when(u + 2 < NU_W)
            def _():
                un = u + 2
                bn = wid * (NU_W // 2) + (un >> 1)
                ln = (un % 2) * UNIT
                for j in range(UCH):
                    pltpu.make_async_copy(
                        ids_hbm.at[bn, pl.ds(ln + j * CHUNK, CHUNK)],
                        idxv[par].at[j], isem).start()
                    pltpu.make_async_copy(
                        tt_hbm.at[bn, pl.ds(ln + j * CHUNK, CHUNK)],
                        ttv[par].at[j], tsem).start()

            bq = wid * (NU_W // 2) + (u >> 1)
            for j in range(UCH):
                jj = par * UCH + j
                for hr in range(HIDDEN // 8):
                    pltpu.make_async_copy(
                        outv.at[pl.ds(jj * HIDDEN + 8 * hr, 8),
                                pl.ds(0, CHUNK)],
                        out_hbm.at[bq, hr, jj], ssem[jj]).start()

        b0 = wid * (NU_W // 2)
        for j in range(UCH):
            pltpu.sync_copy(ids_hbm.at[b0, pl.ds(j * CHUNK, CHUNK)],
                            idxv[0].at[j])
            pltpu.sync_copy(tt_hbm.at[b0, pl.ds(j * CHUNK, CHUNK)],
                            ttv[0].at[j])
        fire_gathers(0)
        for j in range(UCH):
            pltpu.make_async_copy(
                ids_hbm.at[b0, pl.ds(UNIT + j * CHUNK, CHUNK)],
                idxv[1].at[j], isem).start()
            pltpu.make_async_copy(
                tt_hbm.at[b0, pl.ds(UNIT + j * CHUNK, CHUNK)],
                ttv[1].at[j], tsem).start()

        def pair_body(i, c):
            unit_step(i * 2, 0)
            unit_step(i * 2 + 1, 1)
            return c

        lax.fori_loop(0, NU_W // 2, pair_body, 0)

        drain_stores(0)
        drain_stores(1)

    return k


def kernel(input_ids, token_type_ids, word_emb, pos_emb, type_emb, gamma,
           beta):
    ids = input_ids.astype(jnp.int32)
    tt = token_type_ids.astype(jnp.int32)
    gb = jnp.stack([gamma, beta]).astype(jnp.float32)
    mesh = plsc.VectorSubcoreMesh(core_axis_name="c", subcore_axis_name="s")
    out = _ln_rows(mesh)(word_emb, ids, tt, pos_emb, type_emb, gb)
    return out.transpose(0, 2, 4, 1, 3).reshape(B, L, HIDDEN)
